# baseline (speedup 1.0000x reference)
import functools

import jax
import jax.numpy as jnp
from jax import lax
from jax.experimental import pallas as pl
from jax.experimental.pallas import tpu as pltpu
from jax.experimental.pallas import tpu_sc as plsc

_EPS = 1e-12
_NC = 2
_NS = 16
_NW = _NC * _NS
_CH = 128


def _sc_gather(table, idx_flat):
    n = idx_flat.shape[0]
    d = table.shape[1]
    per_w = n // _NW
    n_ch = per_w // _CH
    nbuf = 5
    mesh = plsc.VectorSubcoreMesh(core_axis_name="c", subcore_axis_name="s")

    @functools.partial(
        pl.kernel,
        mesh=mesh,
        out_type=jax.ShapeDtypeStruct((n, d), jnp.float32),
        scratch_types=(
            [pltpu.VMEM((per_w,), jnp.int32)]
            + [pltpu.VMEM((_CH, d), jnp.float32)] * nbuf
            + [pltpu.SemaphoreType.DMA] * (2 * nbuf)
        ),
    )
    def gather_kernel(table_hbm, idx_hbm, out_hbm, idx_v, *rest):
        bufs = rest[:nbuf]
        gsems = rest[nbuf:2 * nbuf]
        ssems = rest[2 * nbuf:]
        wid = lax.axis_index("s") * _NC + lax.axis_index("c")
        base = wid * per_w
        pltpu.sync_copy(idx_hbm.at[pl.ds(base, per_w)], idx_v)

        @pl.loop(0, n_ch // nbuf)
        def _(g):
            handles = []
            for b in range(nbuf):
                i = g * nbuf + b

                @pl.when(g > 0)
                def _():
                    pltpu.make_async_copy(
                        bufs[b], out_hbm.at[pl.ds(0, _CH)], ssems[b]).wait()

                off = pl.multiple_of(i * _CH, _CH)
                handles.append(pltpu.async_copy(
                    table_hbm.at[idx_v.at[pl.ds(off, _CH)]], bufs[b],
                    gsems[b]))
            for b in range(nbuf):
                i = g * nbuf + b
                handles[b].wait()
                off = pl.multiple_of(base + i * _CH, _CH)
                pltpu.async_copy(bufs[b], out_hbm.at[pl.ds(off, _CH)],
                                 ssems[b])

        for b in range(nbuf):
            pltpu.make_async_copy(
                bufs[b], out_hbm.at[pl.ds(0, _CH)], ssems[b]).wait()

    return gather_kernel(table, idx_flat)


def _ln(c, gamma, beta):
    mean = jnp.mean(c, axis=-1, keepdims=True)
    var = jnp.mean((c - mean) ** 2, axis=-1, keepdims=True)
    return (c - mean) / jnp.sqrt(var + _EPS) * gamma + beta


def _tc_fuse(c_word2, eidx3, input_img, img_loc_p, pos_slice, extra_tab,
             img_W, img_b2, loc_W_p, loc_b2, gamma2, beta2, B, S):
    D = pos_slice.shape[1]
    NI = input_img.shape[1]
    VF = input_img.shape[2]
    T = S + NI
    NE = extra_tab.shape[0]
    LP = img_loc_p.shape[2]
    NB = 32

    def body(cw_ref, ei_ref, img_ref, loc_ref, pos_ref, ex_ref,
             W_ref, b_ref, lW_ref, lb_ref, g_ref, be_ref, o_ref):
        gamma = g_ref[...]
        beta = be_ref[...]
        pos = pos_ref[...]
        ex = ex_ref[...].astype(jnp.bfloat16)
        for r in range(NB):
            ei = ei_ref[r]
            ohT = (lax.broadcasted_iota(jnp.int32, (NE, S), 0) == ei
                   ).astype(jnp.bfloat16)
            peg = lax.dot_general(ohT, ex, (((0,), (0,)), ((), ())),
                                  preferred_element_type=jnp.float32)
            c_txt = cw_ref[pl.ds(r * S, S), :] + peg + pos
            o_ref[r, :S, :] = _ln(c_txt, gamma, beta)
        img = img_ref[...].reshape(NB * NI, VF)
        ie = jnp.dot(img, W_ref[...], preferred_element_type=jnp.float32)
        le = jnp.dot(loc_ref[...].reshape(NB * NI, LP), lW_ref[...],
                     preferred_element_type=jnp.float32)
        c_img = (ie + b_ref[...] + le + lb_ref[...]).reshape(NB, NI, D)
        o_ref[:, S:, :] = _ln(c_img, gamma, beta)

    return pl.pallas_call(
        body,
        grid=(B // NB,),
        in_specs=[
            pl.BlockSpec((NB * S, D), lambda i: (i, 0)),
            pl.BlockSpec((NB, 1, S), lambda i: (i, 0, 0)),
            pl.BlockSpec((NB, NI, VF), lambda i: (i, 0, 0)),
            pl.BlockSpec((NB, NI, LP), lambda i: (i, 0, 0)),
            pl.BlockSpec((S, D), lambda i: (0, 0)),
            pl.BlockSpec((NE, D), lambda i: (0, 0)),
            pl.BlockSpec((VF, D), lambda i: (0, 0)),
            pl.BlockSpec((1, D), lambda i: (0, 0)),
            pl.BlockSpec((LP, D), lambda i: (0, 0)),
            pl.BlockSpec((1, D), lambda i: (0, 0)),
            pl.BlockSpec((1, D), lambda i: (0, 0)),
            pl.BlockSpec((1, D), lambda i: (0, 0)),
        ],
        out_specs=pl.BlockSpec((NB, T, D), lambda i: (i, 0, 0)),
        out_shape=jax.ShapeDtypeStruct((B, T, D), jnp.float32),
        compiler_params=pltpu.CompilerParams(
            dimension_semantics=("parallel",)),
    )(c_word2, eidx3, input_img, img_loc_p, pos_slice, extra_tab,
      img_W, img_b2, loc_W_p, loc_b2, gamma2, beta2)


def kernel(input_txt, sentence_pos, input_img, img_loc, token_type_ids,
           word_emb, pos_emb, type_emb, img_W, img_b, loc_W, loc_b,
           ln_gamma, ln_beta, pe):
    B, S = input_txt.shape
    D = word_emb.shape[1]
    c_word2 = _sc_gather(word_emb, input_txt.reshape(B * S))

    n_types = type_emb.shape[0]
    extra_tab = (pe[:, None, :] + type_emb[None, :, :]).reshape(-1, D)
    eidx3 = (sentence_pos * n_types + token_type_ids).reshape(B, 1, S)

    img_loc_p = jnp.pad(img_loc, ((0, 0), (0, 0), (0, 3)))
    loc_W_p = jnp.pad(loc_W, ((0, 3), (0, 0)))
    pos_slice = pos_emb[:S]

    return _tc_fuse(c_word2, eidx3, input_img, img_loc_p, pos_slice,
                    extra_tab, img_W, img_b.reshape(1, D), loc_W_p,
                    loc_b.reshape(1, D), ln_gamma.reshape(1, D),
                    ln_beta.reshape(1, D), B, S)


# speedup vs baseline: 1.0183x; 1.0183x over previous
import functools

import jax
import jax.numpy as jnp
from jax import lax
from jax.experimental import pallas as pl
from jax.experimental.pallas import tpu as pltpu
from jax.experimental.pallas import tpu_sc as plsc

_EPS = 1e-12
_NC = 2
_NS = 16
_NW = _NC * _NS
_CH = 128


def _sc_gather(table, idx_flat):
    n = idx_flat.shape[0]
    d = table.shape[1]
    per_w = n // _NW
    n_ch = per_w // _CH
    nbuf = 5
    mesh = plsc.VectorSubcoreMesh(core_axis_name="c", subcore_axis_name="s")

    @functools.partial(
        pl.kernel,
        mesh=mesh,
        out_type=jax.ShapeDtypeStruct((n, d), jnp.float32),
        scratch_types=(
            [pltpu.VMEM((per_w,), jnp.int32)]
            + [pltpu.VMEM((_CH, d), jnp.float32)] * nbuf
            + [pltpu.SemaphoreType.DMA] * (2 * nbuf)
        ),
    )
    def gather_kernel(table_hbm, idx_hbm, out_hbm, idx_v, *rest):
        bufs = rest[:nbuf]
        gsems = rest[nbuf:2 * nbuf]
        ssems = rest[2 * nbuf:]
        wid = lax.axis_index("s") * _NC + lax.axis_index("c")
        base = wid * per_w
        pltpu.sync_copy(idx_hbm.at[pl.ds(base, per_w)], idx_v)

        @pl.loop(0, n_ch // nbuf)
        def _(g):
            handles = []
            for b in range(nbuf):
                i = g * nbuf + b

                @pl.when(g > 0)
                def _():
                    pltpu.make_async_copy(
                        bufs[b], out_hbm.at[pl.ds(0, _CH)], ssems[b]).wait()

                off = pl.multiple_of(i * _CH, _CH)
                handles.append(pltpu.async_copy(
                    table_hbm.at[idx_v.at[pl.ds(off, _CH)]], bufs[b],
                    gsems[b]))
            for b in range(nbuf):
                i = g * nbuf + b
                handles[b].wait()
                off = pl.multiple_of(base + i * _CH, _CH)
                pltpu.async_copy(bufs[b], out_hbm.at[pl.ds(off, _CH)],
                                 ssems[b])

        for b in range(nbuf):
            pltpu.make_async_copy(
                bufs[b], out_hbm.at[pl.ds(0, _CH)], ssems[b]).wait()

    return gather_kernel(table, idx_flat)


def _ln(c, gamma, beta):
    mean = jnp.mean(c, axis=-1, keepdims=True)
    var = jnp.mean((c - mean) ** 2, axis=-1, keepdims=True)
    return (c - mean) / jnp.sqrt(var + _EPS) * gamma + beta


def _tc_fuse(c_word2, eidx3, input_img, img_loc_p, pos_slice, extra_tab,
             img_W, img_b2, loc_W_p, loc_b2, gamma2, beta2, B, S):
    D = pos_slice.shape[1]
    NI = input_img.shape[1]
    VF = input_img.shape[2]
    T = S + NI
    NE = extra_tab.shape[0]
    LP = img_loc_p.shape[2]
    NB = 32

    def body(cw_ref, ei_ref, img_ref, loc_ref, pos_ref, ex_ref,
             W_ref, b_ref, lW_ref, lb_ref, g_ref, be_ref, o_ref):
        gamma = g_ref[...]
        beta = be_ref[...]
        pos = pos_ref[...]
        ex = ex_ref[...]
        for r in range(NB):
            ei = ei_ref[r]
            ohT = (lax.broadcasted_iota(jnp.int32, (NE, S), 0) == ei
                   ).astype(jnp.float32)
            peg = lax.dot_general(ohT, ex, (((0,), (0,)), ((), ())),
                                  preferred_element_type=jnp.float32)
            c_txt = cw_ref[pl.ds(r * S, S), :] + peg + pos
            o_ref[r, :S, :] = _ln(c_txt, gamma, beta)
        img = img_ref[...].reshape(NB * NI, VF)
        ie = jnp.dot(img, W_ref[...], preferred_element_type=jnp.float32)
        le = jnp.dot(loc_ref[...].reshape(NB * NI, LP), lW_ref[...],
                     preferred_element_type=jnp.float32)
        c_img = (ie + b_ref[...] + le + lb_ref[...]).reshape(NB, NI, D)
        o_ref[:, S:, :] = _ln(c_img, gamma, beta)

    return pl.pallas_call(
        body,
        grid=(B // NB,),
        in_specs=[
            pl.BlockSpec((NB * S, D), lambda i: (i, 0)),
            pl.BlockSpec((NB, 1, S), lambda i: (i, 0, 0)),
            pl.BlockSpec((NB, NI, VF), lambda i: (i, 0, 0)),
            pl.BlockSpec((NB, NI, LP), lambda i: (i, 0, 0)),
            pl.BlockSpec((S, D), lambda i: (0, 0)),
            pl.BlockSpec((NE, D), lambda i: (0, 0)),
            pl.BlockSpec((VF, D), lambda i: (0, 0)),
            pl.BlockSpec((1, D), lambda i: (0, 0)),
            pl.BlockSpec((LP, D), lambda i: (0, 0)),
            pl.BlockSpec((1, D), lambda i: (0, 0)),
            pl.BlockSpec((1, D), lambda i: (0, 0)),
            pl.BlockSpec((1, D), lambda i: (0, 0)),
        ],
        out_specs=pl.BlockSpec((NB, T, D), lambda i: (i, 0, 0)),
        out_shape=jax.ShapeDtypeStruct((B, T, D), jnp.float32),
        compiler_params=pltpu.CompilerParams(
            dimension_semantics=("parallel",)),
    )(c_word2, eidx3, input_img, img_loc_p, pos_slice, extra_tab,
      img_W, img_b2, loc_W_p, loc_b2, gamma2, beta2)


def kernel(input_txt, sentence_pos, input_img, img_loc, token_type_ids,
           word_emb, pos_emb, type_emb, img_W, img_b, loc_W, loc_b,
           ln_gamma, ln_beta, pe):
    B, S = input_txt.shape
    D = word_emb.shape[1]
    c_word2 = _sc_gather(word_emb, input_txt.reshape(B * S))

    n_types = type_emb.shape[0]
    extra_tab = (pe[:, None, :] + type_emb[None, :, :]).reshape(-1, D)
    eidx3 = (sentence_pos * n_types + token_type_ids).reshape(B, 1, S)

    img_loc_p = jnp.pad(img_loc, ((0, 0), (0, 0), (0, 3)))
    loc_W_p = jnp.pad(loc_W, ((0, 3), (0, 0)))
    pos_slice = pos_emb[:S]

    return _tc_fuse(c_word2, eidx3, input_img, img_loc_p, pos_slice,
                    extra_tab, img_W, img_b.reshape(1, D), loc_W_p,
                    loc_b.reshape(1, D), ln_gamma.reshape(1, D),
                    ln_beta.reshape(1, D), B, S)
